# Initial kernel scaffold; baseline (speedup 1.0000x reference)
import jax
import jax.numpy as jnp
from jax import lax
from jax.experimental import pallas as pl
from jax.experimental.pallas import tpu as pltpu
from jax.experimental.pallas import tpu_sc as plsc

_TEMPERATURE = 0.07
_MARGIN = 0.3
_B = 1024
_D = 64
_RB = 128
_RB2 = 8
_IB = 128
_NC = 2
_NS = 16
_NW = _NC * _NS
_L = 16


def _stats_body(a_ref, c_ref, pm_ref, ks_ref, kn_ref, loss_ref, acc_ref,
                cnt_ref):
    step = pl.program_id(0)
    a = a_ref[...]
    c = c_ref[...]
    pm = pm_ref[...]
    an = a / jnp.maximum(jnp.sqrt(jnp.sum(a * a, axis=1, keepdims=True)),
                         1e-12)
    cn = c / jnp.maximum(jnp.sqrt(jnp.sum(c * c, axis=1, keepdims=True)),
                         1e-12)
    sim = lax.dot_general(an, cn, (((1,), (1,)), ((), ())),
                          preferred_element_type=jnp.float32)
    cnt = jnp.sum(pm, axis=1, keepdims=True)
    pos = jnp.sum(sim * pm, axis=1, keepdims=True) / jnp.maximum(cnt, 1.0)
    negm = pm == 0.0
    semi = jnp.logical_and(sim > pos - _MARGIN, negm)
    neg_inf = jnp.float32(-jnp.inf)
    ks_ref[...] = jnp.where(semi, sim, neg_inf)
    kn_ref[...] = jnp.where(negm, sim, neg_inf)
    row_max = jnp.max(sim, axis=1, keepdims=True)
    m = jnp.maximum(row_max, pos)
    se = (jnp.sum(jnp.exp((sim - m) / _TEMPERATURE), axis=1, keepdims=True)
          + jnp.exp((pos - m) / _TEMPERATURE))
    loss_rows = m / _TEMPERATURE + jnp.log(se) - pos / _TEMPERATURE
    acc_rows = (pos >= row_max).astype(jnp.float32)
    lsum = jnp.sum(loss_rows)
    asum = jnp.sum(acc_rows)
    ssum = jnp.sum(semi.astype(jnp.float32))

    @pl.when(step == 0)
    def _():
        loss_ref[0, 0] = lsum
        acc_ref[0, 0] = asum
        cnt_ref[0, 0] = ssum

    @pl.when(step != 0)
    def _():
        loss_ref[0, 0] += lsum
        acc_ref[0, 0] += asum
        cnt_ref[0, 0] += ssum


def _mine_stats(anchors, candidates, pm_f):
    return pl.pallas_call(
        _stats_body,
        grid=(_B // _RB,),
        in_specs=[
            pl.BlockSpec((_RB, _D), lambda i: (i, 0)),
            pl.BlockSpec((_B, _D), lambda i: (0, 0)),
            pl.BlockSpec((_RB, _B), lambda i: (i, 0)),
        ],
        out_specs=[
            pl.BlockSpec((_RB, _B), lambda i: (i, 0)),
            pl.BlockSpec((_RB, _B), lambda i: (i, 0)),
            pl.BlockSpec(memory_space=pltpu.SMEM),
            pl.BlockSpec(memory_space=pltpu.SMEM),
            pl.BlockSpec(memory_space=pltpu.SMEM),
        ],
        out_shape=[
            jax.ShapeDtypeStruct((_B, _B), jnp.float32),
            jax.ShapeDtypeStruct((_B, _B), jnp.float32),
            jax.ShapeDtypeStruct((1, 1), jnp.float32),
            jax.ShapeDtypeStruct((1, 1), jnp.float32),
            jax.ShapeDtypeStruct((1, 1), jnp.float32),
        ],
    )(anchors, candidates, pm_f)


def _rank_body(cnt_ref, ksi_ref, ksj_ref, kni_ref, knj_ref, out_ref):
    ib = pl.program_id(1)
    use_semi = cnt_ref[0, 0] > 0.0
    ki = jnp.where(use_semi, ksi_ref[...], kni_ref[...])
    kj = jnp.where(use_semi, ksj_ref[...], knj_ref[...])
    ki3 = ki[:, :, None]
    kj3 = kj[:, None, :]
    ii = lax.broadcasted_iota(jnp.int32, (_IB, _B), 0) + ib * _IB
    jj = lax.broadcasted_iota(jnp.int32, (_IB, _B), 1)
    tie = (ii < jj)[None, :, :]
    cmp = jnp.where(tie, ki3 >= kj3, ki3 > kj3)
    partial = jnp.sum(cmp.astype(jnp.int32), axis=1)

    @pl.when(ib == 0)
    def _():
        out_ref[...] = partial

    @pl.when(ib != 0)
    def _():
        out_ref[...] += partial


def _rank(cnt_s, ks, kn):
    return pl.pallas_call(
        _rank_body,
        grid=(_B // _RB2, _B // _IB),
        in_specs=[
            pl.BlockSpec(memory_space=pltpu.SMEM),
            pl.BlockSpec((_RB2, _IB), lambda b, i: (b, i)),
            pl.BlockSpec((_RB2, _B), lambda b, i: (b, 0)),
            pl.BlockSpec((_RB2, _IB), lambda b, i: (b, i)),
            pl.BlockSpec((_RB2, _B), lambda b, i: (b, 0)),
        ],
        out_specs=pl.BlockSpec((_RB2, _B), lambda b, i: (b, 0)),
        out_shape=jax.ShapeDtypeStruct((_B, _B), jnp.int32),
    )(cnt_s, ks, kn, ks, kn)


def _invert_body(rank_hbm, out_hbm, row_v, inv_v):
    wid = lax.axis_index("s") * _NC + lax.axis_index("c")
    rows_per = _B // _NW

    def row_step(r, carry):
        row = wid * rows_per + r
        pltpu.sync_copy(rank_hbm.at[row], row_v)

        def chunk(k, c2):
            idx = row_v[pl.ds(k * _L, _L)]
            vals = lax.broadcasted_iota(jnp.int32, (_L,), 0) + k * _L
            plsc.store_scatter(inv_v, [idx], vals)
            return c2

        lax.fori_loop(0, _B // _L, chunk, 0)
        pltpu.sync_copy(inv_v, out_hbm.at[row])
        return carry

    lax.fori_loop(0, rows_per, row_step, 0)


def _invert(rank):
    f = pl.kernel(
        _invert_body,
        mesh=plsc.VectorSubcoreMesh(core_axis_name="c", subcore_axis_name="s"),
        out_type=jax.ShapeDtypeStruct((_B, _B), jnp.int32),
        scratch_types=[
            pltpu.VMEM((_B,), jnp.int32),
            pltpu.VMEM((_B,), jnp.int32),
        ],
    )
    return f(rank)


def kernel(anchors, candidates, positive_mask):
    pm_f = positive_mask.astype(jnp.float32)
    ks, kn, loss_s, acc_s, cnt_s = _mine_stats(anchors, candidates, pm_f)
    rank = _rank(cnt_s, ks, kn)
    hard_indices = _invert(rank)
    loss = loss_s[0, 0] / _B
    accuracy = acc_s[0, 0] / _B
    return loss, accuracy, hard_indices


# speedup vs baseline: 3.0091x; 3.0091x over previous
import jax
import jax.numpy as jnp
from jax import lax
from jax.experimental import pallas as pl
from jax.experimental.pallas import tpu as pltpu
from jax.experimental.pallas import tpu_sc as plsc

_TEMPERATURE = 0.07
_MARGIN = 0.3
_B = 1024
_D = 64
_RB = 128
_RB2 = 8
_IB = 128
_NC = 2
_NS = 16
_NW = _NC * _NS
_L = 16


def _stats_body(a_ref, c_ref, pm_ref, ks_ref, kn_ref, loss_ref, acc_ref,
                cnt_ref):
    step = pl.program_id(0)
    a = a_ref[...]
    c = c_ref[...]
    pm = pm_ref[...]
    an = a / jnp.maximum(jnp.sqrt(jnp.sum(a * a, axis=1, keepdims=True)),
                         1e-12)
    cn = c / jnp.maximum(jnp.sqrt(jnp.sum(c * c, axis=1, keepdims=True)),
                         1e-12)
    sim = lax.dot_general(an, cn, (((1,), (1,)), ((), ())),
                          preferred_element_type=jnp.float32)
    cnt = jnp.sum(pm, axis=1, keepdims=True)
    pos = jnp.sum(sim * pm, axis=1, keepdims=True) / jnp.maximum(cnt, 1.0)
    negm = pm == 0.0
    semi = jnp.logical_and(sim > pos - _MARGIN, negm)
    neg_inf = jnp.float32(-jnp.inf)
    ks_ref[...] = jnp.where(semi, sim, neg_inf)
    kn_ref[...] = jnp.where(negm, sim, neg_inf)
    row_max = jnp.max(sim, axis=1, keepdims=True)
    m = jnp.maximum(row_max, pos)
    se = (jnp.sum(jnp.exp((sim - m) / _TEMPERATURE), axis=1, keepdims=True)
          + jnp.exp((pos - m) / _TEMPERATURE))
    loss_rows = m / _TEMPERATURE + jnp.log(se) - pos / _TEMPERATURE
    acc_rows = (pos >= row_max).astype(jnp.float32)
    lsum = jnp.sum(loss_rows)
    asum = jnp.sum(acc_rows)
    ssum = jnp.sum(semi.astype(jnp.float32))

    @pl.when(step == 0)
    def _():
        loss_ref[0, 0] = lsum
        acc_ref[0, 0] = asum
        cnt_ref[0, 0] = ssum

    @pl.when(step != 0)
    def _():
        loss_ref[0, 0] += lsum
        acc_ref[0, 0] += asum
        cnt_ref[0, 0] += ssum


def _mine_stats(anchors, candidates, pm_f):
    return pl.pallas_call(
        _stats_body,
        grid=(_B // _RB,),
        in_specs=[
            pl.BlockSpec((_RB, _D), lambda i: (i, 0)),
            pl.BlockSpec((_B, _D), lambda i: (0, 0)),
            pl.BlockSpec((_RB, _B), lambda i: (i, 0)),
        ],
        out_specs=[
            pl.BlockSpec((_RB, _B), lambda i: (i, 0)),
            pl.BlockSpec((_RB, _B), lambda i: (i, 0)),
            pl.BlockSpec(memory_space=pltpu.SMEM),
            pl.BlockSpec(memory_space=pltpu.SMEM),
            pl.BlockSpec(memory_space=pltpu.SMEM),
        ],
        out_shape=[
            jax.ShapeDtypeStruct((_B, _B), jnp.float32),
            jax.ShapeDtypeStruct((_B, _B), jnp.float32),
            jax.ShapeDtypeStruct((1, 1), jnp.float32),
            jax.ShapeDtypeStruct((1, 1), jnp.float32),
            jax.ShapeDtypeStruct((1, 1), jnp.float32),
        ],
    )(anchors, candidates, pm_f)


def _rank_body(cnt_ref, ksi_ref, ksj_ref, kni_ref, knj_ref, out_ref):
    ib = pl.program_id(1)
    use_semi = cnt_ref[0, 0] > 0.0
    ki = jnp.where(use_semi, ksi_ref[...], kni_ref[...])
    kj = jnp.where(use_semi, ksj_ref[...], knj_ref[...])
    ki3 = ki[:, :, None]
    kj3 = kj[:, None, :]
    one = jnp.float32(1.0)
    zero = jnp.float32(0.0)
    ii = lax.broadcasted_iota(jnp.int32, (_IB, _B), 0) + ib * _IB
    jj = lax.broadcasted_iota(jnp.int32, (_IB, _B), 1)
    tie_f = jnp.where(ii < jj, one, zero)[None, :, :]
    gt_f = jnp.where(ki3 > kj3, one, zero)
    eq_f = jnp.where(ki3 == kj3, one, zero)
    cmp_f = gt_f + tie_f * eq_f
    partial = jnp.sum(cmp_f, axis=1).astype(jnp.int32)

    @pl.when(ib == 0)
    def _():
        out_ref[...] = partial

    @pl.when(ib != 0)
    def _():
        out_ref[...] += partial


def _rank(cnt_s, ks, kn):
    return pl.pallas_call(
        _rank_body,
        grid=(_B // _RB2, _B // _IB),
        in_specs=[
            pl.BlockSpec(memory_space=pltpu.SMEM),
            pl.BlockSpec((_RB2, _IB), lambda b, i: (b, i)),
            pl.BlockSpec((_RB2, _B), lambda b, i: (b, 0)),
            pl.BlockSpec((_RB2, _IB), lambda b, i: (b, i)),
            pl.BlockSpec((_RB2, _B), lambda b, i: (b, 0)),
        ],
        out_specs=pl.BlockSpec((_RB2, _B), lambda b, i: (b, 0)),
        out_shape=jax.ShapeDtypeStruct((_B, _B), jnp.int32),
    )(cnt_s, ks, ks, kn, kn)


def _invert_body(rank_hbm, out_hbm, row_v, inv_v):
    wid = lax.axis_index("s") * _NC + lax.axis_index("c")
    rows_per = _B // _NW

    def row_step(r, carry):
        row = wid * rows_per + r
        pltpu.sync_copy(rank_hbm.at[row], row_v)

        def chunk(k, c2):
            idx = row_v[pl.ds(k * _L, _L)]
            vals = lax.broadcasted_iota(jnp.int32, (_L,), 0) + k * _L
            plsc.store_scatter(inv_v, [idx], vals)
            return c2

        lax.fori_loop(0, _B // _L, chunk, 0)
        pltpu.sync_copy(inv_v, out_hbm.at[row])
        return carry

    lax.fori_loop(0, rows_per, row_step, 0)


def _invert(rank):
    f = pl.kernel(
        _invert_body,
        mesh=plsc.VectorSubcoreMesh(core_axis_name="c", subcore_axis_name="s"),
        out_type=jax.ShapeDtypeStruct((_B, _B), jnp.int32),
        scratch_types=[
            pltpu.VMEM((_B,), jnp.int32),
            pltpu.VMEM((_B,), jnp.int32),
        ],
        compiler_params=pltpu.CompilerParams(needs_layout_passes=False),
    )
    return f(rank)


def kernel(anchors, candidates, positive_mask):
    pm_f = positive_mask.astype(jnp.float32)
    ks, kn, loss_s, acc_s, cnt_s = _mine_stats(anchors, candidates, pm_f)
    rank = _rank(cnt_s, ks, kn)
    hard_indices = _invert(rank)
    loss = loss_s[0, 0] / _B
    accuracy = acc_s[0, 0] / _B
    return loss, accuracy, hard_indices


# speedup vs baseline: 5.9938x; 1.9919x over previous
import jax
import jax.numpy as jnp
from jax import lax
from jax.experimental import pallas as pl
from jax.experimental.pallas import tpu as pltpu
from jax.experimental.pallas import tpu_sc as plsc

_TEMPERATURE = 0.07
_MARGIN = 0.3
_B = 1024
_D = 64
_RB = 128
_RB2 = 8
_IB = 128
_NC = 2
_NS = 16
_NW = _NC * _NS
_L = 16


def _stats_body(a_ref, c_ref, pm_ref, ks_ref, kn_ref, loss_ref, acc_ref,
                cnt_ref):
    step = pl.program_id(0)
    a = a_ref[...]
    c = c_ref[...]
    pm = pm_ref[...]
    an = a / jnp.maximum(jnp.sqrt(jnp.sum(a * a, axis=1, keepdims=True)),
                         1e-12)
    cn = c / jnp.maximum(jnp.sqrt(jnp.sum(c * c, axis=1, keepdims=True)),
                         1e-12)
    sim = lax.dot_general(an, cn, (((1,), (1,)), ((), ())),
                          preferred_element_type=jnp.float32)
    cnt = jnp.sum(pm, axis=1, keepdims=True)
    pos = jnp.sum(sim * pm, axis=1, keepdims=True) / jnp.maximum(cnt, 1.0)
    negm = pm == 0.0
    semi = jnp.logical_and(sim > pos - _MARGIN, negm)
    neg_inf = jnp.float32(-jnp.inf)
    ks_ref[...] = jnp.where(semi, sim, neg_inf)
    kn_ref[...] = jnp.where(negm, sim, neg_inf)
    row_max = jnp.max(sim, axis=1, keepdims=True)
    m = jnp.maximum(row_max, pos)
    se = (jnp.sum(jnp.exp((sim - m) / _TEMPERATURE), axis=1, keepdims=True)
          + jnp.exp((pos - m) / _TEMPERATURE))
    loss_rows = m / _TEMPERATURE + jnp.log(se) - pos / _TEMPERATURE
    acc_rows = (pos >= row_max).astype(jnp.float32)
    lsum = jnp.sum(loss_rows)
    asum = jnp.sum(acc_rows)
    ssum = jnp.sum(semi.astype(jnp.float32))

    @pl.when(step == 0)
    def _():
        loss_ref[0, 0] = lsum
        acc_ref[0, 0] = asum
        cnt_ref[0, 0] = ssum

    @pl.when(step != 0)
    def _():
        loss_ref[0, 0] += lsum
        acc_ref[0, 0] += asum
        cnt_ref[0, 0] += ssum


def _mine_stats(anchors, candidates, pm_f):
    return pl.pallas_call(
        _stats_body,
        grid=(_B // _RB,),
        in_specs=[
            pl.BlockSpec((_RB, _D), lambda i: (i, 0)),
            pl.BlockSpec((_B, _D), lambda i: (0, 0)),
            pl.BlockSpec((_RB, _B), lambda i: (i, 0)),
        ],
        out_specs=[
            pl.BlockSpec((_RB, _B), lambda i: (i, 0)),
            pl.BlockSpec((_RB, _B), lambda i: (i, 0)),
            pl.BlockSpec(memory_space=pltpu.SMEM),
            pl.BlockSpec(memory_space=pltpu.SMEM),
            pl.BlockSpec(memory_space=pltpu.SMEM),
        ],
        out_shape=[
            jax.ShapeDtypeStruct((_B, _B), jnp.float32),
            jax.ShapeDtypeStruct((_B, _B), jnp.float32),
            jax.ShapeDtypeStruct((1, 1), jnp.float32),
            jax.ShapeDtypeStruct((1, 1), jnp.float32),
            jax.ShapeDtypeStruct((1, 1), jnp.float32),
        ],
    )(anchors, candidates, pm_f)


def _rank_body(cnt_ref, ksj_ref, knj_ref, out_ref):
    use_semi = cnt_ref[0, 0] > 0.0
    kj = jnp.where(use_semi, ksj_ref[...], knj_ref[...])
    one = jnp.float32(1.0)
    zero = jnp.float32(0.0)
    ii = lax.broadcasted_iota(jnp.int32, (_IB, _IB), 0)
    jj = lax.broadcasted_iota(jnp.int32, (_IB, _IB), 1)
    tie_f = jnp.where(ii < jj, one, zero)[None, :, :]
    tot = None
    for ib in range(_B // _IB):
        lo = ib * _IB
        hi = lo + _IB
        ki3 = kj[:, lo:hi, None]
        pieces = []
        if lo > 0:
            left = kj[:, None, :lo]
            pieces.append(jnp.sum(jnp.where(ki3 > left, one, zero), axis=1))
        diag = kj[:, None, lo:hi]
        d = (jnp.where(ki3 > diag, one, zero)
             + tie_f * jnp.where(ki3 == diag, one, zero))
        pieces.append(jnp.sum(d, axis=1))
        if hi < _B:
            right = kj[:, None, hi:]
            pieces.append(jnp.sum(jnp.where(ki3 >= right, one, zero), axis=1))
        contrib = jnp.concatenate(pieces, axis=1)
        tot = contrib if tot is None else tot + contrib
    out_ref[...] = tot.astype(jnp.int32)


def _rank(cnt_s, ks, kn):
    return pl.pallas_call(
        _rank_body,
        grid=(_B // _RB2,),
        in_specs=[
            pl.BlockSpec(memory_space=pltpu.SMEM),
            pl.BlockSpec((_RB2, _B), lambda b: (b, 0)),
            pl.BlockSpec((_RB2, _B), lambda b: (b, 0)),
        ],
        out_specs=pl.BlockSpec((_RB2, _B), lambda b: (b, 0)),
        out_shape=jax.ShapeDtypeStruct((_B, _B), jnp.int32),
    )(cnt_s, ks, kn)


def _invert_body(rank_hbm, out_hbm, row_v, inv_v):
    wid = lax.axis_index("s") * _NC + lax.axis_index("c")
    rows_per = _B // _NW

    def row_step(r, carry):
        row = wid * rows_per + r
        pltpu.sync_copy(rank_hbm.at[row], row_v)

        def chunk(k, c2):
            idx = row_v[pl.ds(k * _L, _L)]
            vals = lax.broadcasted_iota(jnp.int32, (_L,), 0) + k * _L
            plsc.store_scatter(inv_v, [idx], vals)
            return c2

        lax.fori_loop(0, _B // _L, chunk, 0)
        pltpu.sync_copy(inv_v, out_hbm.at[row])
        return carry

    lax.fori_loop(0, rows_per, row_step, 0)


def _invert(rank):
    f = pl.kernel(
        _invert_body,
        mesh=plsc.VectorSubcoreMesh(core_axis_name="c", subcore_axis_name="s"),
        out_type=jax.ShapeDtypeStruct((_B, _B), jnp.int32),
        scratch_types=[
            pltpu.VMEM((_B,), jnp.int32),
            pltpu.VMEM((_B,), jnp.int32),
        ],
        compiler_params=pltpu.CompilerParams(needs_layout_passes=False),
    )
    return f(rank)


def kernel(anchors, candidates, positive_mask):
    pm_f = positive_mask.astype(jnp.float32)
    ks, kn, loss_s, acc_s, cnt_s = _mine_stats(anchors, candidates, pm_f)
    rank = _rank(cnt_s, ks, kn)
    hard_indices = _invert(rank)
    loss = loss_s[0, 0] / _B
    accuracy = acc_s[0, 0] / _B
    return loss, accuracy, hard_indices
